# baseline (speedup 1.0000x reference)
import jax
import jax.numpy as jnp
from jax.experimental import pallas as pl
from jax.experimental.pallas import tpu as pltpu

_BPB = 2


def _gcn_body(adj_ref, feat_ref, wp_ref, out_ref):
    adj = adj_ref[...]
    feat = feat_ref[...]
    s = adj.shape[-1]
    ft = feat.shape[-1]
    w1 = wp_ref[0:ft, :]
    b1 = wp_ref[ft:ft + 1, :]
    w2 = wp_ref[ft + 1:ft + 1 + w1.shape[1], :]
    b2 = wp_ref[ft + 1 + w1.shape[1]:ft + 2 + w1.shape[1], :]
    deg = jnp.maximum(jnp.sum(adj, axis=2, keepdims=True), 1.0)
    invdeg = 1.0 / deg
    g = jax.lax.dot_general(feat, w1, (((2,), (0,)), ((), ())),
                            preferred_element_type=jnp.float32)
    m = jax.lax.dot_general(adj.astype(jnp.bfloat16), g.astype(jnp.bfloat16),
                            (((2,), (1,)), ((0,), (0,))),
                            preferred_element_type=jnp.float32)
    h = jnp.maximum(m * invdeg + b1, 0.0)
    c = jax.lax.dot_general(invdeg, adj, (((1,), (1,)), ((0,), (0,))),
                            preferred_element_type=jnp.float32)
    y = jax.lax.dot_general(c, h, (((2,), (1,)), ((0,), (0,))),
                            preferred_element_type=jnp.float32) * (1.0 / s)
    y = jax.lax.dot_general(y, w2, (((2,), (0,)), ((), ())),
                            preferred_element_type=jnp.float32) + b2
    nrm = jnp.sqrt(jnp.sum(y * y, axis=-1, keepdims=True))
    out_ref[...] = y / jnp.maximum(nrm, 1e-5)


@jax.jit
def kernel(adj, n_feat, W1, b1, W2, b2):
    B, S, _ = adj.shape
    FT = n_feat.shape[-1]
    H = W1.shape[-1]
    O = W2.shape[-1]
    wpack = jnp.concatenate(
        [W1, b1.reshape(1, H), W2, b2.reshape(1, O)], axis=0)
    return pl.pallas_call(
        _gcn_body,
        grid=(B // _BPB,),
        in_specs=[
            pl.BlockSpec((_BPB, S, S), lambda b: (b, 0, 0)),
            pl.BlockSpec((_BPB, S, FT), lambda b: (b, 0, 0)),
            pl.BlockSpec(wpack.shape, lambda b: (0, 0)),
        ],
        out_specs=pl.BlockSpec((_BPB, 1, O), lambda b: (b, 0, 0)),
        out_shape=jax.ShapeDtypeStruct((B, 1, O), jnp.float32),
        compiler_params=pltpu.CompilerParams(
            dimension_semantics=("parallel",),
            vmem_limit_bytes=120 * 1024 * 1024),
    )(adj, n_feat, wpack).reshape(B, O)


# speedup vs baseline: 1.0168x; 1.0168x over previous
import jax
import jax.numpy as jnp
from jax.experimental import pallas as pl
from jax.experimental.pallas import tpu as pltpu

_BPB = 4


def _gcn_body(adj_ref, feat_ref, wp_ref, out_ref):
    adj = adj_ref[...]
    feat = feat_ref[...]
    s = adj.shape[-1]
    ft = feat.shape[-1]
    w1 = wp_ref[0:ft, :]
    b1 = wp_ref[ft:ft + 1, :]
    w2 = wp_ref[ft + 1:ft + 1 + w1.shape[1], :]
    b2 = wp_ref[ft + 1 + w1.shape[1]:ft + 2 + w1.shape[1], :]
    deg = jnp.maximum(jnp.sum(adj, axis=2, keepdims=True), 1.0)
    invdeg = 1.0 / deg
    g = jax.lax.dot_general(feat, w1, (((2,), (0,)), ((), ())),
                            preferred_element_type=jnp.float32)
    m = jax.lax.dot_general(adj.astype(jnp.bfloat16), g.astype(jnp.bfloat16),
                            (((2,), (1,)), ((0,), (0,))),
                            preferred_element_type=jnp.float32)
    h = jnp.maximum(m * invdeg + b1, 0.0)
    c = jax.lax.dot_general(invdeg, adj, (((1,), (1,)), ((0,), (0,))),
                            preferred_element_type=jnp.float32)
    y = jax.lax.dot_general(c, h, (((2,), (1,)), ((0,), (0,))),
                            preferred_element_type=jnp.float32) * (1.0 / s)
    y = jax.lax.dot_general(y, w2, (((2,), (0,)), ((), ())),
                            preferred_element_type=jnp.float32) + b2
    nrm = jnp.sqrt(jnp.sum(y * y, axis=-1, keepdims=True))
    out_ref[...] = y / jnp.maximum(nrm, 1e-5)


@jax.jit
def kernel(adj, n_feat, W1, b1, W2, b2):
    B, S, _ = adj.shape
    FT = n_feat.shape[-1]
    H = W1.shape[-1]
    O = W2.shape[-1]
    wpack = jnp.concatenate(
        [W1, b1.reshape(1, H), W2, b2.reshape(1, O)], axis=0)
    return pl.pallas_call(
        _gcn_body,
        grid=(B // _BPB,),
        in_specs=[
            pl.BlockSpec((_BPB, S, S), lambda b: (b, 0, 0)),
            pl.BlockSpec((_BPB, S, FT), lambda b: (b, 0, 0)),
            pl.BlockSpec(wpack.shape, lambda b: (0, 0)),
        ],
        out_specs=pl.BlockSpec((_BPB, 1, O), lambda b: (b, 0, 0)),
        out_shape=jax.ShapeDtypeStruct((B, 1, O), jnp.float32),
        compiler_params=pltpu.CompilerParams(
            dimension_semantics=("parallel",),
            vmem_limit_bytes=120 * 1024 * 1024),
    )(adj, n_feat, wpack).reshape(B, O)
